# baseline (speedup 1.0000x reference)
import functools

import jax
import jax.numpy as jnp
from jax import lax
from jax.experimental import pallas as pl
from jax.experimental.pallas import tpu as pltpu
from jax.experimental.pallas import tpu_sc as plsc

_N = 10000
_E = 320000
_D = 128
_H = 16

_NC = 2
_NS = 16
_NW = _NC * _NS

_EPC = _E // _NC
_EPT = _E // _NW
_CHUNK = 2000
_NCHUNK = _EPT // _CHUNK
_SCHUNK = 2000
_NSCHUNK = _EPT // _SCHUNK

_NP = 10240
_RPT = _NP // _NS
_DPT = _NP // _NS
_G = _RPT // _H

_scat_scratch = [
    pltpu.VMEM((_SCHUNK,), jnp.int32),
    pltpu.VMEM((_SCHUNK,), jnp.int32),
    pltpu.VMEM((_SCHUNK, _H), jnp.float32),
    pltpu.VMEM((_SCHUNK,), jnp.int32),
    pltpu.VMEM((_SCHUNK,), jnp.int32),
    pltpu.VMEM((_SCHUNK, _H), jnp.float32),
    pltpu.SemaphoreType.DMA,
    pltpu.SemaphoreType.DMA,
    pltpu.SemaphoreType.DMA,
    pltpu.SemaphoreType.DMA,
    pltpu.SemaphoreType.DMA,
    pltpu.SemaphoreType.DMA,
    pltpu.VMEM((_RPT, _H), jnp.float32),
    pltpu.VMEM((_RPT, _H), jnp.float32),
    pltpu.VMEM((_DPT,), jnp.float32),
    pltpu.VMEM((_DPT,), jnp.float32),
]


def _rsqrt16(x):
    i = lax.bitcast_convert_type(x, jnp.int32)
    i = 0x5F3759DF - lax.shift_right_logical(i, 1)
    y = lax.bitcast_convert_type(i, jnp.float32)
    for _ in range(3):
        y = y * (1.5 - 0.5 * x * y * y)
    return y


def _edge_pipeline(gather_ref, src_hbm, dst_hbm, acc_sh, bufs, cid, sid):
    def start_idx(c):
        s_v, d_v = bufs[c % 2][0], bufs[c % 2][1]
        sem = bufs[c % 2][3]
        base = cid * _EPC + sid * _EPT + c * _SCHUNK
        c1 = pltpu.async_copy(src_hbm.at[pl.ds(base, _SCHUNK)], s_v, sem)
        c2 = pltpu.async_copy(dst_hbm.at[pl.ds(base, _SCHUNK)], d_v, sem)
        return (c1, c2)

    pend_idx = [None, None]
    pend_scat = [None, None]
    pend_idx[0] = start_idx(0)
    for c in range(_NSCHUNK):
        b = c % 2
        s_v, d_v, r_v, _, sem_g, sem_s = bufs[b]
        for d in pend_idx[b]:
            d.wait()
        gat = pltpu.async_copy(gather_ref.at[s_v], r_v, sem_g)
        if c + 1 < _NSCHUNK:
            if pend_scat[1 - b] is not None:
                pend_scat[1 - b].wait()
                pend_scat[1 - b] = None
            pend_idx[1 - b] = start_idx(c + 1)
        gat.wait()
        pend_scat[b] = pltpu.async_copy(r_v, acc_sh.at[d_v], sem_s, add=True)
    for b in range(2):
        if pend_scat[b] is not None:
            pend_scat[b].wait()


_mesh = plsc.VectorSubcoreMesh(core_axis_name="c", subcore_axis_name="s")

_EPTH = _E // _NS
_NHCHUNK = _EPTH // _SCHUNK


@functools.partial(
    pl.kernel,
    out_type=(
        jax.ShapeDtypeStruct((_NC, _NP, _H), jnp.float32),
        jax.ShapeDtypeStruct((_NP,), jnp.float32),
    ),
    mesh=_mesh,
    scratch_types=_scat_scratch + [
        pltpu.VMEM((_SCHUNK,), jnp.float32),
        pltpu.VMEM_SHARED((_NP,), jnp.float32),
        pltpu.VMEM_SHARED((_NP, _H), jnp.float32),
        pltpu.VMEM_SHARED((_NP, _H), jnp.float32),
    ],
    compiler_params=pltpu.CompilerParams(use_tc_tiling_on_sc=False),
)
def _sc_pass1(hp_hbm, src_hbm, dst_hbm, out_hbm, degf_hbm,
              src0, dst0, rows0, src1, dst1, rows1,
              sem_i0, sem_g0, sem_s0, sem_i1, sem_g1, sem_s1,
              sA, sB, dg0, dg1, ones_v, deg_sh, u1_sh, acc_sh):
    cid = lax.axis_index("c")
    sid = lax.axis_index("s")
    bufs = ((src0, dst0, rows0, sem_i0, sem_g0, sem_s0),
            (src1, dst1, rows1, sem_i1, sem_g1, sem_s1))
    sl = pl.ds(sid * _RPT, _RPT)
    dsl = pl.ds(sid * _DPT, _DPT)

    ch = pltpu.async_copy(hp_hbm.at[sl], sA, sem_i0)

    one = jnp.full((_H,), 1.0, jnp.float32)
    zero = jnp.zeros((_H,), jnp.float32)
    for i in range(_DPT // _H):
        dg1[pl.ds(i * _H, _H)] = zero
    pltpu.sync_copy(dg1, deg_sh.at[dsl])
    for i in range(_SCHUNK // _H):
        ones_v[pl.ds(i * _H, _H)] = one
    plsc.subcore_barrier()

    hbufs = ((src0, sem_g0, sem_s0), (src1, sem_g1, sem_s1))

    def start_hidx(c):
        d_v, sem, _ = hbufs[c % 2]
        base = sid * _EPTH + c * _SCHUNK
        return pltpu.async_copy(dst_hbm.at[pl.ds(base, _SCHUNK)], d_v, sem)

    pend_idx = [None, None]
    pend_scat = [None, None]
    pend_idx[0] = start_hidx(0)
    for c in range(_NHCHUNK):
        b = c % 2
        d_v, _, sem_s = hbufs[b]
        pend_idx[b].wait()
        if c + 1 < _NHCHUNK:
            if pend_scat[1 - b] is not None:
                pend_scat[1 - b].wait()
                pend_scat[1 - b] = None
            pend_idx[1 - b] = start_hidx(c + 1)
        pend_scat[b] = pltpu.async_copy(ones_v, deg_sh.at[d_v], sem_s, add=True)
    for b in range(2):
        if pend_scat[b] is not None:
            pend_scat[b].wait()
    plsc.subcore_barrier()

    ch.wait()
    pltpu.sync_copy(deg_sh.at[dsl], dg0)

    def pre_body(g, _):
        deg = dg0[pl.ds(g * _H, _H)] + 1.0
        y = _rsqrt16(deg)
        for l in range(_H):
            i = g * _H + l
            row = sA[i, :] * y[l]
            sA[i, :] = row
            sB[i, :] = row * 0.5
        return 0

    lax.fori_loop(0, _G, pre_body, 0)
    pltpu.sync_copy(sA, u1_sh.at[sl])
    pltpu.sync_copy(sB, acc_sh.at[sl])
    pltpu.sync_copy(deg_sh.at[dsl], degf_hbm.at[dsl])
    plsc.subcore_barrier()

    _edge_pipeline(u1_sh, src_hbm, dst_hbm, acc_sh, bufs, cid, sid)

    plsc.subcore_barrier()
    pltpu.sync_copy(acc_sh.at[sl], out_hbm.at[cid, sl])


@functools.partial(
    pl.kernel,
    out_type=jax.ShapeDtypeStruct((_NC, _NP, _H), jnp.float32),
    mesh=_mesh,
    scratch_types=_scat_scratch + [
        pltpu.VMEM((_H,), jnp.float32),
        pltpu.VMEM_SHARED((_NP, _H), jnp.float32),
        pltpu.VMEM_SHARED((_NP, _H), jnp.float32),
    ],
    compiler_params=pltpu.CompilerParams(use_tc_tiling_on_sc=False),
)
def _sc_pass2(s1p_hbm, degf_hbm, b1_hbm, src_hbm, dst_hbm, out_hbm,
              src0, dst0, rows0, src1, dst1, rows1,
              sem_i0, sem_g0, sem_s0, sem_i1, sem_g1, sem_s1,
              sA, sB, dg0, dg1, b1v, u2_sh, acc_sh):
    cid = lax.axis_index("c")
    sid = lax.axis_index("s")
    bufs = ((src0, dst0, rows0, sem_i0, sem_g0, sem_s0),
            (src1, dst1, rows1, sem_i1, sem_g1, sem_s1))
    sl = pl.ds(sid * _RPT, _RPT)

    c1 = pltpu.async_copy(s1p_hbm.at[0, sl], sA, sem_i0)
    c2 = pltpu.async_copy(s1p_hbm.at[1, sl], sB, sem_i0)
    c3 = pltpu.async_copy(degf_hbm.at[pl.ds(sid * _DPT, _DPT)], dg0, sem_i0)
    c4 = pltpu.async_copy(b1_hbm, b1v, sem_i0)
    for c in (c1, c2, c3, c4):
        c.wait()
    b1row = b1v[...]

    def mid_body(g, _):
        deg = dg0[pl.ds(g * _H, _H)] + 1.0
        y = _rsqrt16(deg)
        for l in range(_H):
            i = g * _H + l
            s = y[l]
            u2 = (sA[i, :] + sB[i, :]) * (s * s) + b1row * s
            sA[i, :] = u2
            sB[i, :] = u2 * 0.5
        return 0

    lax.fori_loop(0, _G, mid_body, 0)
    pltpu.sync_copy(sA, u2_sh.at[sl])
    pltpu.sync_copy(sB, acc_sh.at[sl])
    plsc.subcore_barrier()

    _edge_pipeline(u2_sh, src_hbm, dst_hbm, acc_sh, bufs, cid, sid)

    plsc.subcore_barrier()
    pltpu.sync_copy(acc_sh.at[sl], out_hbm.at[cid, sl])


def _tc_mm1_body(x_ref, w_ref, o_ref):
    o_ref[0:_N, :] = jnp.dot(x_ref[...], w_ref[...],
                             preferred_element_type=jnp.float32)
    o_ref[_N:_NP, :] = jnp.zeros((_NP - _N, _H), jnp.float32)


def _tc_final_body(s2_ref, degf_ref, w2_ref, b2_ref, o_ref):
    deg = jnp.reshape(degf_ref[...], (1, _NP))[:, 0:_N] + 1.0
    dinv = jnp.transpose(lax.rsqrt(deg), (1, 0))
    m2 = (s2_ref[0, 0:_N, :] + s2_ref[1, 0:_N, :]) * dinv
    o_ref[...] = (
        jnp.dot(m2, w2_ref[...], preferred_element_type=jnp.float32)
        + jnp.reshape(b2_ref[...], (1, _D))
    )


def kernel(x, edge_index, W1, b1, W2, b2):
    src = edge_index[0]
    dst = edge_index[1]

    hp = pl.pallas_call(
        _tc_mm1_body,
        out_shape=jax.ShapeDtypeStruct((_NP, _H), jnp.float32),
    )(x, W1)

    s1p, degf = _sc_pass1(hp, src, dst)

    s2p = _sc_pass2(s1p, degf, b1, src, dst)

    out = pl.pallas_call(
        _tc_final_body,
        out_shape=jax.ShapeDtypeStruct((_N, _D), jnp.float32),
    )(s2p, degf, W2, b2)
    return out


# speedup vs baseline: 1.0172x; 1.0172x over previous
import functools

import jax
import jax.numpy as jnp
from jax import lax
from jax.experimental import pallas as pl
from jax.experimental.pallas import tpu as pltpu
from jax.experimental.pallas import tpu_sc as plsc

_N = 10000
_E = 320000
_D = 128
_H = 16

_NC = 2
_NS = 16
_NW = _NC * _NS

_EPC = _E // _NC
_EPT = _E // _NW
_CHUNK = 2000
_NCHUNK = _EPT // _CHUNK
_SCHUNK = 2000
_NSCHUNK = _EPT // _SCHUNK

_NP = 10240
_RPT = _NP // _NS
_DPT = _NP // _NS
_G = _RPT // _H

_scat_scratch = [
    pltpu.VMEM((_SCHUNK,), jnp.int32),
    pltpu.VMEM((_SCHUNK,), jnp.int32),
    pltpu.VMEM((_SCHUNK, _H), jnp.float32),
    pltpu.VMEM((_SCHUNK,), jnp.int32),
    pltpu.VMEM((_SCHUNK,), jnp.int32),
    pltpu.VMEM((_SCHUNK, _H), jnp.float32),
    pltpu.SemaphoreType.DMA,
    pltpu.SemaphoreType.DMA,
    pltpu.SemaphoreType.DMA,
    pltpu.SemaphoreType.DMA,
    pltpu.SemaphoreType.DMA,
    pltpu.SemaphoreType.DMA,
    pltpu.VMEM((_RPT, _H), jnp.float32),
    pltpu.VMEM((_RPT, _H), jnp.float32),
    pltpu.VMEM((_DPT,), jnp.float32),
    pltpu.VMEM((_DPT,), jnp.float32),
]


def _rsqrt16(x):
    i = lax.bitcast_convert_type(x, jnp.int32)
    i = 0x5F3759DF - lax.shift_right_logical(i, 1)
    y = lax.bitcast_convert_type(i, jnp.float32)
    for _ in range(3):
        y = y * (1.5 - 0.5 * x * y * y)
    return y


def _start_edge_idx(src_hbm, dst_hbm, bufs, cid, sid, c):
    s_v, d_v = bufs[c % 2][0], bufs[c % 2][1]
    sem = bufs[c % 2][3]
    base = cid * _EPC + sid * _EPT + c * _SCHUNK
    c1 = pltpu.async_copy(src_hbm.at[pl.ds(base, _SCHUNK)], s_v, sem)
    c2 = pltpu.async_copy(dst_hbm.at[pl.ds(base, _SCHUNK)], d_v, sem)
    return (c1, c2)


def _edge_pipeline(gather_ref, src_hbm, dst_hbm, acc_sh, bufs, cid, sid,
                   pre0=None):
    def start_idx(c):
        return _start_edge_idx(src_hbm, dst_hbm, bufs, cid, sid, c)

    pend_idx = [None, None]
    pend_scat = [None, None]
    pend_idx[0] = pre0 if pre0 is not None else start_idx(0)
    for c in range(_NSCHUNK):
        b = c % 2
        s_v, d_v, r_v, _, sem_g, sem_s = bufs[b]
        for d in pend_idx[b]:
            d.wait()
        gat = pltpu.async_copy(gather_ref.at[s_v], r_v, sem_g)
        if c + 1 < _NSCHUNK:
            if pend_scat[1 - b] is not None:
                pend_scat[1 - b].wait()
                pend_scat[1 - b] = None
            pend_idx[1 - b] = start_idx(c + 1)
        gat.wait()
        pend_scat[b] = pltpu.async_copy(r_v, acc_sh.at[d_v], sem_s, add=True)
    for b in range(2):
        if pend_scat[b] is not None:
            pend_scat[b].wait()


_mesh = plsc.VectorSubcoreMesh(core_axis_name="c", subcore_axis_name="s")

_EPTH = _E // _NS
_NHCHUNK = _EPTH // _SCHUNK


@functools.partial(
    pl.kernel,
    out_type=(
        jax.ShapeDtypeStruct((_NC, _NP, _H), jnp.float32),
        jax.ShapeDtypeStruct((_NP,), jnp.float32),
    ),
    mesh=_mesh,
    scratch_types=_scat_scratch + [
        pltpu.VMEM((_SCHUNK,), jnp.float32),
        pltpu.VMEM_SHARED((_NP,), jnp.float32),
        pltpu.VMEM_SHARED((_NP, _H), jnp.float32),
        pltpu.VMEM_SHARED((_NP, _H), jnp.float32),
    ],
    compiler_params=pltpu.CompilerParams(use_tc_tiling_on_sc=False),
)
def _sc_pass1(hp_hbm, src_hbm, dst_hbm, out_hbm, degf_hbm,
              src0, dst0, rows0, src1, dst1, rows1,
              sem_i0, sem_g0, sem_s0, sem_i1, sem_g1, sem_s1,
              sA, sB, dg0, dg1, ones_v, deg_sh, u1_sh, acc_sh):
    cid = lax.axis_index("c")
    sid = lax.axis_index("s")
    bufs = ((src0, dst0, rows0, sem_i0, sem_g0, sem_s0),
            (src1, dst1, rows1, sem_i1, sem_g1, sem_s1))
    sl = pl.ds(sid * _RPT, _RPT)
    dsl = pl.ds(sid * _DPT, _DPT)

    ch = pltpu.async_copy(hp_hbm.at[sl], sA, sem_i0)

    one = jnp.full((_H,), 1.0, jnp.float32)
    zero = jnp.zeros((_H,), jnp.float32)
    for i in range(_DPT // _H):
        dg1[pl.ds(i * _H, _H)] = zero
    pltpu.sync_copy(dg1, deg_sh.at[dsl])
    for i in range(_SCHUNK // _H):
        ones_v[pl.ds(i * _H, _H)] = one
    plsc.subcore_barrier()

    hbufs = ((src0, sem_g0, sem_s0), (src1, sem_g1, sem_s1))

    def start_hidx(c):
        d_v, sem, _ = hbufs[c % 2]
        base = sid * _EPTH + c * _SCHUNK
        return pltpu.async_copy(dst_hbm.at[pl.ds(base, _SCHUNK)], d_v, sem)

    pend_idx = [None, None]
    pend_scat = [None, None]
    pend_idx[0] = start_hidx(0)
    for c in range(_NHCHUNK):
        b = c % 2
        d_v, _, sem_s = hbufs[b]
        pend_idx[b].wait()
        if c + 1 < _NHCHUNK:
            if pend_scat[1 - b] is not None:
                pend_scat[1 - b].wait()
                pend_scat[1 - b] = None
            pend_idx[1 - b] = start_hidx(c + 1)
        pend_scat[b] = pltpu.async_copy(ones_v, deg_sh.at[d_v], sem_s, add=True)
    for b in range(2):
        if pend_scat[b] is not None:
            pend_scat[b].wait()
    plsc.subcore_barrier()

    pre0 = _start_edge_idx(src_hbm, dst_hbm, bufs, cid, sid, 0)

    ch.wait()
    pltpu.sync_copy(deg_sh.at[dsl], dg0)

    def pre_body(g, _):
        deg = dg0[pl.ds(g * _H, _H)] + 1.0
        y = _rsqrt16(deg)
        for l in range(_H):
            i = g * _H + l
            row = sA[i, :] * y[l]
            sA[i, :] = row
            sB[i, :] = row * 0.5
        return 0

    lax.fori_loop(0, _G, pre_body, 0)
    pltpu.sync_copy(sA, u1_sh.at[sl])
    pltpu.sync_copy(sB, acc_sh.at[sl])
    pltpu.sync_copy(deg_sh.at[dsl], degf_hbm.at[dsl])
    plsc.subcore_barrier()

    _edge_pipeline(u1_sh, src_hbm, dst_hbm, acc_sh, bufs, cid, sid, pre0=pre0)

    plsc.subcore_barrier()
    pltpu.sync_copy(acc_sh.at[sl], out_hbm.at[cid, sl])


@functools.partial(
    pl.kernel,
    out_type=jax.ShapeDtypeStruct((_NC, _NP, _H), jnp.float32),
    mesh=_mesh,
    scratch_types=_scat_scratch + [
        pltpu.VMEM((_H,), jnp.float32),
        pltpu.VMEM_SHARED((_NP, _H), jnp.float32),
        pltpu.VMEM_SHARED((_NP, _H), jnp.float32),
    ],
    compiler_params=pltpu.CompilerParams(use_tc_tiling_on_sc=False),
)
def _sc_pass2(s1p_hbm, degf_hbm, b1_hbm, src_hbm, dst_hbm, out_hbm,
              src0, dst0, rows0, src1, dst1, rows1,
              sem_i0, sem_g0, sem_s0, sem_i1, sem_g1, sem_s1,
              sA, sB, dg0, dg1, b1v, u2_sh, acc_sh):
    cid = lax.axis_index("c")
    sid = lax.axis_index("s")
    bufs = ((src0, dst0, rows0, sem_i0, sem_g0, sem_s0),
            (src1, dst1, rows1, sem_i1, sem_g1, sem_s1))
    sl = pl.ds(sid * _RPT, _RPT)

    pre0 = _start_edge_idx(src_hbm, dst_hbm, bufs, cid, sid, 0)

    c1 = pltpu.async_copy(s1p_hbm.at[0, sl], sA, sem_i0)
    c2 = pltpu.async_copy(s1p_hbm.at[1, sl], sB, sem_i0)
    c3 = pltpu.async_copy(degf_hbm.at[pl.ds(sid * _DPT, _DPT)], dg0, sem_i0)
    c4 = pltpu.async_copy(b1_hbm, b1v, sem_i0)
    for c in (c1, c2, c3, c4):
        c.wait()
    b1row = b1v[...]

    def mid_body(g, _):
        deg = dg0[pl.ds(g * _H, _H)] + 1.0
        y = _rsqrt16(deg)
        for l in range(_H):
            i = g * _H + l
            s = y[l]
            u2 = (sA[i, :] + sB[i, :]) * (s * s) + b1row * s
            sA[i, :] = u2
            sB[i, :] = u2 * 0.5
        return 0

    lax.fori_loop(0, _G, mid_body, 0)
    pltpu.sync_copy(sA, u2_sh.at[sl])
    pltpu.sync_copy(sB, acc_sh.at[sl])
    plsc.subcore_barrier()

    _edge_pipeline(u2_sh, src_hbm, dst_hbm, acc_sh, bufs, cid, sid, pre0=pre0)

    plsc.subcore_barrier()
    pltpu.sync_copy(acc_sh.at[sl], out_hbm.at[cid, sl])


def _tc_mm1_body(x_ref, w_ref, o_ref):
    o_ref[0:_N, :] = jnp.dot(x_ref[...], w_ref[...],
                             preferred_element_type=jnp.float32)
    o_ref[_N:_NP, :] = jnp.zeros((_NP - _N, _H), jnp.float32)


def _tc_final_body(s2_ref, degf_ref, w2_ref, b2_ref, o_ref):
    deg = jnp.reshape(degf_ref[...], (1, _NP))[:, 0:_N] + 1.0
    dinv = jnp.transpose(lax.rsqrt(deg), (1, 0))
    m2 = (s2_ref[0, 0:_N, :] + s2_ref[1, 0:_N, :]) * dinv
    o_ref[...] = (
        jnp.dot(m2, w2_ref[...], preferred_element_type=jnp.float32)
        + jnp.reshape(b2_ref[...], (1, _D))
    )


def kernel(x, edge_index, W1, b1, W2, b2):
    src = edge_index[0]
    dst = edge_index[1]

    hp = pl.pallas_call(
        _tc_mm1_body,
        out_shape=jax.ShapeDtypeStruct((_NP, _H), jnp.float32),
    )(x, W1)

    s1p, degf = _sc_pass1(hp, src, dst)

    s2p = _sc_pass2(s1p, degf, b1, src, dst)

    out = pl.pallas_call(
        _tc_final_body,
        out_shape=jax.ShapeDtypeStruct((_N, _D), jnp.float32),
    )(s2p, degf, W2, b2)
    return out
